# Initial kernel scaffold; baseline (speedup 1.0000x reference)
import jax
import jax.numpy as jnp
from jax.experimental import pallas as pl


def kernel(x, targets, table):
    raise NotImplementedError("write your pallas kernel here")


# speedup vs baseline: 1.8156x; 1.8156x over previous
import functools

import jax
import jax.numpy as jnp
from jax import lax
from jax.experimental import pallas as pl
from jax.experimental.pallas import tpu as pltpu
from jax.experimental.pallas import tpu_sc as plsc

V = 1000
BT = 4096 * 20
NC = 2
NS = 16
L = 16
NW = NC * NS
B_PER_W = BT // NW
K = 32
CHUNKS = B_PER_W // K


def _lse_body(t_ref, out_ref):
    t = t_ref[...]
    m = jnp.max(t, axis=1, keepdims=True)
    s = jnp.sum(jnp.exp(t - m), axis=1, keepdims=True)
    out_ref[...] = m + jnp.log(s)


_lse_call = pl.pallas_call(
    _lse_body,
    out_shape=jax.ShapeDtypeStruct((V, 1), jnp.float32),
)


def _fin_body(p_ref, o_ref):
    o_ref[...] = (jnp.sum(p_ref[...]) * (1.0 / BT)).reshape(1, 1)


_fin_call = pl.pallas_call(
    _fin_body,
    out_shape=jax.ShapeDtypeStruct((1, 1), jnp.float32),
)


_mesh = plsc.VectorSubcoreMesh(core_axis_name="c", subcore_axis_name="s")


@functools.partial(
    pl.kernel,
    out_type=(
        jax.ShapeDtypeStruct((BT, V), jnp.float32),
        jax.ShapeDtypeStruct((NW, L), jnp.float32),
    ),
    mesh=_mesh,
    compiler_params=pltpu.CompilerParams(
        use_tc_tiling_on_sc=False, needs_layout_passes=False),
    scratch_types=[
        pltpu.VMEM((B_PER_W,), jnp.int32),
        pltpu.VMEM((B_PER_W,), jnp.int32),
        pltpu.VMEM((V,), jnp.float32),
        pltpu.VMEM((K, V), jnp.float32),
        pltpu.VMEM((L,), jnp.float32),
        pltpu.SemaphoreType.DMA,
    ],
)
def _sc_gather(x_hbm, tgt_hbm, lse_hbm, table_hbm, out_hbm, part_hbm,
               xloc, tgtloc, lseloc, rows, accv, sem):
    wid = lax.axis_index("s") * NC + lax.axis_index("c")
    base = wid * B_PER_W
    pltpu.sync_copy(x_hbm.at[pl.ds(base, B_PER_W)], xloc)
    pltpu.sync_copy(tgt_hbm.at[pl.ds(base, B_PER_W)], tgtloc)
    pltpu.sync_copy(lse_hbm, lseloc)

    def chunk(g, acc):
        r0 = g * K
        pltpu.async_copy(table_hbm.at[xloc.at[pl.ds(r0, K)]], rows, sem).wait()
        pltpu.sync_copy(rows, out_hbm.at[pl.ds(base + r0, K)])
        for j in range(K // L):
            ids16 = lax.iota(jnp.int32, L) + (j * L)
            xv = xloc[pl.ds(r0 + j * L, L)]
            tv = tgtloc[pl.ds(r0 + j * L, L)]
            picked = plsc.load_gather(rows, [ids16, tv])
            lsev = plsc.load_gather(lseloc, [xv])
            acc = acc + (lsev - picked)
        return acc

    acc = lax.fori_loop(0, CHUNKS, chunk, jnp.zeros((L,), jnp.float32))
    accv[...] = acc
    pltpu.sync_copy(accv, part_hbm.at[wid])


def kernel(x, targets, table):
    xf = x.reshape(BT).astype(jnp.int32)
    tf = targets.reshape(BT).astype(jnp.int32)
    lse = _lse_call(table).reshape(V)
    logits, part = _sc_gather(xf, tf, lse, table)
    loss = _fin_call(part)[0, 0]
    return logits, loss
